# baseline (speedup 1.0000x reference)
import functools

import jax
import jax.numpy as jnp
from jax import lax
from jax.experimental import pallas as pl
from jax.experimental.pallas import tpu as pltpu
from jax.experimental.pallas import tpu_sc as plsc

B = 16384
D = 32
L = 16
NC = 2
NS = 16
NW = NC * NS
BPW = B // NW
CHUNK = 128
NCHUNK = BPW // CHUNK


def _recommender_body(user_hbm, article_hbm, cust_hbm, art_hbm, out_hbm,
                      uidx_v, aidx_v, crow_v, arow_v, out_v, sem):
    wid = lax.axis_index("s") * NC + lax.axis_index("c")
    base = wid * BPW

    pltpu.sync_copy(user_hbm.at[wid], uidx_v)
    pltpu.sync_copy(article_hbm.at[wid], aidx_v)

    copies = []
    for j in range(NCHUNK):
        copies.append(pltpu.async_copy(
            cust_hbm.at[uidx_v.at[j]], crow_v.at[pl.ds(j * CHUNK, CHUNK)],
            sem))
        copies.append(pltpu.async_copy(
            art_hbm.at[aidx_v.at[j]], arow_v.at[pl.ds(j * CHUNK, CHUNK)],
            sem))
    for c in copies:
        c.wait()

    lane = lax.iota(jnp.int32, L)

    def group(g, carry):
        row0 = g * L
        acc = jnp.zeros((L,), jnp.float32)
        for j in range(L):
            i = row0 + j
            q = (crow_v[i, pl.ds(0, L)] * arow_v[i, pl.ds(0, L)] +
                 crow_v[i, pl.ds(L, L)] * arow_v[i, pl.ds(L, L)])
            acc = jnp.where(lane == j, jnp.sum(q), acc)
        out_v[pl.ds(row0, L)] = acc
        return carry

    lax.fori_loop(0, BPW // L, group, 0)

    pltpu.sync_copy(out_v, out_hbm.at[pl.ds(base, BPW)])


def kernel(user, article, customer_embed, articles_embed):
    mesh = plsc.VectorSubcoreMesh(core_axis_name="c", subcore_axis_name="s")
    k = functools.partial(
        pl.kernel,
        mesh=mesh,
        out_type=jax.ShapeDtypeStruct((B,), jnp.float32),
        scratch_types=[
            pltpu.VMEM((NCHUNK, CHUNK), jnp.int32),
            pltpu.VMEM((NCHUNK, CHUNK), jnp.int32),
            pltpu.VMEM((BPW, D), jnp.float32),
            pltpu.VMEM((BPW, D), jnp.float32),
            pltpu.VMEM((BPW,), jnp.float32),
            pltpu.SemaphoreType.DMA,
        ],
        compiler_params=pltpu.CompilerParams(
            needs_layout_passes=False, use_tc_tiling_on_sc=False),
    )(_recommender_body)
    user3 = user.reshape(NW, NCHUNK, CHUNK)
    article3 = article.reshape(NW, NCHUNK, CHUNK)
    return k(user3, article3, customer_embed, articles_embed)


# speedup vs baseline: 2.1362x; 2.1362x over previous
import functools

import jax
import jax.numpy as jnp
from jax import lax
from jax.experimental import pallas as pl
from jax.experimental.pallas import tpu as pltpu
from jax.experimental.pallas import tpu_sc as plsc

B = 16384
D = 32
L = 16
NW = 32
BPW = B // NW
W = 128
CHUNK = 8
NCHUNK = BPW // CHUNK


def _dot_body(user_hbm, article_hbm, ct_hbm, at_hbm, out_hbm,
              uidx_v, aidx_v, cwin_v, awin_v, out_v, sem):
    wid = lax.axis_index("s") * 2 + lax.axis_index("c")
    base = wid * BPW

    pltpu.sync_copy(user_hbm.at[pl.ds(base, BPW)], uidx_v.at[pl.ds(0, BPW)])
    pltpu.sync_copy(article_hbm.at[pl.ds(base, BPW)], aidx_v.at[pl.ds(0, BPW)])

    lanes = lax.iota(jnp.int32, L)

    def chunk_body(c, acc):
        cbase = c * CHUNK
        uvec = uidx_v[pl.ds(cbase, L)]
        avec = aidx_v[pl.ds(cbase, L)]
        copies = []
        for j in range(CHUNK):
            ub = pl.multiple_of((uvec[j] // W) * W, W)
            ab = pl.multiple_of((avec[j] // W) * W, W)
            copies.append(pltpu.async_copy(
                ct_hbm.at[:, pl.ds(ub, W)], cwin_v.at[j], sem))
            copies.append(pltpu.async_copy(
                at_hbm.at[:, pl.ds(ab, W)], awin_v.at[j], sem))
        for cp in copies:
            cp.wait()

        half = (c & 1) * CHUNK
        for j in range(CHUNK):
            cu = uvec[j] % W
            ca = avec[j] % W
            jv = jnp.full((L,), j, jnp.int32)
            c0 = plsc.load_gather(
                cwin_v, [jv, lanes, jnp.full((L,), cu, jnp.int32)])
            c1 = plsc.load_gather(
                cwin_v, [jv, lanes + L, jnp.full((L,), cu, jnp.int32)])
            a0 = plsc.load_gather(
                awin_v, [jv, lanes, jnp.full((L,), ca, jnp.int32)])
            a1 = plsc.load_gather(
                awin_v, [jv, lanes + L, jnp.full((L,), ca, jnp.int32)])
            q = c0 * a0 + c1 * a1
            acc = jnp.where(lanes == half + j, jnp.sum(q), acc)

        @pl.when((c & 1) == 1)
        def _():
            out_v[pl.ds((c >> 1) * L, L)] = acc

        return acc

    lax.fori_loop(0, NCHUNK, chunk_body, jnp.zeros((L,), jnp.float32))

    pltpu.sync_copy(out_v, out_hbm.at[pl.ds(base, BPW)])


def kernel(user, article, customer_embed, articles_embed):
    mesh = plsc.VectorSubcoreMesh(core_axis_name="c", subcore_axis_name="s")
    k = functools.partial(
        pl.kernel,
        mesh=mesh,
        out_type=jax.ShapeDtypeStruct((B,), jnp.float32),
        scratch_types=[
            pltpu.VMEM((BPW + CHUNK,), jnp.int32),
            pltpu.VMEM((BPW + CHUNK,), jnp.int32),
            pltpu.VMEM((CHUNK, D, W), jnp.float32),
            pltpu.VMEM((CHUNK, D, W), jnp.float32),
            pltpu.VMEM((BPW,), jnp.float32),
            pltpu.SemaphoreType.DMA,
        ],
        compiler_params=pltpu.CompilerParams(
            needs_layout_passes=False, use_tc_tiling_on_sc=True),
    )(_dot_body)
    return k(user, article, customer_embed.T, articles_embed.T)


# speedup vs baseline: 2.4603x; 1.1517x over previous
import functools

import jax
import jax.numpy as jnp
from jax import lax
from jax.experimental import pallas as pl
from jax.experimental.pallas import tpu as pltpu
from jax.experimental.pallas import tpu_sc as plsc

B = 16384
D = 32
L = 16
NW = 32
BPW = B // NW
IC = 128
NIC = BPW // IC
W = 128
CHUNK = 8
NCHUNK = BPW // CHUNK


def _gather_body(article_hbm, at_hbm, out_hbm, aidx_v, rows_v, sem):
    wid = lax.axis_index("s") * 2 + lax.axis_index("c")
    pltpu.sync_copy(article_hbm.at[wid], aidx_v)
    copies = []
    for j in range(NIC):
        copies.append(pltpu.async_copy(
            at_hbm.at[aidx_v.at[j]], rows_v.at[pl.ds(j * IC, IC)], sem))
    for cp in copies:
        cp.wait()
    pltpu.sync_copy(rows_v, out_hbm.at[pl.ds(wid * BPW, BPW)])


def _dot_body(user_hbm, ct_hbm, arows_hbm, out_hbm,
              uidx_v, arow_v, cwin_v, out_v, sem):
    wid = lax.axis_index("s") * 2 + lax.axis_index("c")
    base = wid * BPW

    pltpu.sync_copy(user_hbm.at[pl.ds(base, BPW)], uidx_v.at[pl.ds(0, BPW)])
    pltpu.sync_copy(arows_hbm.at[pl.ds(base * D, BPW * D)], arow_v)

    lanes = lax.iota(jnp.int32, L)

    def chunk_body(c, acc):
        cbase = c * CHUNK
        uvec = uidx_v[pl.ds(cbase, L)]
        copies = []
        for j in range(CHUNK):
            ub = pl.multiple_of((uvec[j] // W) * W, W)
            copies.append(pltpu.async_copy(
                ct_hbm.at[:, pl.ds(ub, W)], cwin_v.at[j], sem))
        for cp in copies:
            cp.wait()

        half = (c & 1) * CHUNK
        for j in range(CHUNK):
            cu = uvec[j] % W
            jv = jnp.full((L,), j, jnp.int32)
            c0 = plsc.load_gather(
                cwin_v, [jv, lanes, jnp.full((L,), cu, jnp.int32)])
            c1 = plsc.load_gather(
                cwin_v, [jv, lanes + L, jnp.full((L,), cu, jnp.int32)])
            a0 = arow_v[pl.ds((cbase + j) * D, L)]
            a1 = arow_v[pl.ds((cbase + j) * D + L, L)]
            q = c0 * a0 + c1 * a1
            acc = jnp.where(lanes == half + j, jnp.sum(q), acc)

        @pl.when((c & 1) == 1)
        def _():
            out_v[pl.ds((c >> 1) * L, L)] = acc

        return acc

    lax.fori_loop(0, NCHUNK, chunk_body, jnp.zeros((L,), jnp.float32))

    pltpu.sync_copy(out_v, out_hbm.at[pl.ds(base, BPW)])


def kernel(user, article, customer_embed, articles_embed):
    mesh = plsc.VectorSubcoreMesh(core_axis_name="c", subcore_axis_name="s")
    gather = functools.partial(
        pl.kernel,
        mesh=mesh,
        out_type=jax.ShapeDtypeStruct((B, D), jnp.float32),
        scratch_types=[
            pltpu.VMEM((NIC, IC), jnp.int32),
            pltpu.VMEM((BPW, D), jnp.float32),
            pltpu.SemaphoreType.DMA,
        ],
        compiler_params=pltpu.CompilerParams(
            needs_layout_passes=False, use_tc_tiling_on_sc=False),
    )(_gather_body)
    art_rows = gather(article.reshape(NW, NIC, IC), articles_embed)

    dot = functools.partial(
        pl.kernel,
        mesh=mesh,
        out_type=jax.ShapeDtypeStruct((B,), jnp.float32),
        scratch_types=[
            pltpu.VMEM((BPW + L,), jnp.int32),
            pltpu.VMEM((BPW * D,), jnp.float32),
            pltpu.VMEM((CHUNK, D, W), jnp.float32),
            pltpu.VMEM((BPW,), jnp.float32),
            pltpu.SemaphoreType.DMA,
        ],
        compiler_params=pltpu.CompilerParams(
            needs_layout_passes=False, use_tc_tiling_on_sc=True),
    )(_dot_body)
    return dot(user, customer_embed.T, art_rows.reshape(B * D))
